# baseline (device time: 100951 ns/iter reference)
import jax
from jax import lax
from jax.experimental import pallas as pl
from jax.experimental.pallas import tpu as pltpu


def kernel(x, pi):
    def body(x_ref, pi_ref, out_ref, send_sem, recv_sem):
        my_x = lax.axis_index("x")
        my_y = lax.axis_index("y")
        my_z = lax.axis_index("z")
        dst_y = pi_ref[my_y]

        barrier_sem = pltpu.get_barrier_semaphore()
        pl.semaphore_signal(
            barrier_sem,
            inc=1,
            device_id=(my_x, 1 - my_y, my_z),
            device_id_type=pl.DeviceIdType.MESH,
        )
        pl.semaphore_wait(barrier_sem, 1)

        @pl.when(dst_y == my_y)
        def _():
            copy = pltpu.make_async_copy(x_ref, out_ref, send_sem.at[0])
            copy.start()
            copy.wait()

        @pl.when(dst_y != my_y)
        def _():
            m_half = x_ref.shape[1] // 2
            rdmas = []
            for c in range(2):
                sl = pl.ds(c * m_half, m_half)
                rdma = pltpu.make_async_remote_copy(
                    src_ref=x_ref.at[:, sl],
                    dst_ref=out_ref.at[:, sl],
                    send_sem=send_sem.at[c],
                    recv_sem=recv_sem.at[c],
                    device_id=(my_x, dst_y, my_z),
                    device_id_type=pl.DeviceIdType.MESH,
                )
                rdma.start()
                rdmas.append(rdma)
            for rdma in rdmas:
                rdma.wait()

    return pl.pallas_call(
        body,
        out_shape=jax.ShapeDtypeStruct(x.shape, x.dtype),
        in_specs=[
            pl.BlockSpec(memory_space=pl.ANY),
            pl.BlockSpec(memory_space=pltpu.SMEM),
        ],
        out_specs=pl.BlockSpec(memory_space=pl.ANY),
        scratch_shapes=[
            pltpu.SemaphoreType.DMA((2,)),
            pltpu.SemaphoreType.DMA((2,)),
        ],
        compiler_params=pltpu.CompilerParams(collective_id=0),
    )(x, pi)


# device time: 100922 ns/iter; 1.0003x vs baseline; 1.0003x over previous
import jax
from jax import lax
from jax.experimental import pallas as pl
from jax.experimental.pallas import tpu as pltpu


def kernel(x, pi):
    def body(x_ref, pi_ref, out_ref, send_sem, recv_sem):
        my_x = lax.axis_index("x")
        my_y = lax.axis_index("y")
        my_z = lax.axis_index("z")
        dst_y = pi_ref[my_y]

        barrier_sem = pltpu.get_barrier_semaphore()
        pl.semaphore_signal(
            barrier_sem,
            inc=1,
            device_id=(my_x, 1 - my_y, my_z),
            device_id_type=pl.DeviceIdType.MESH,
        )
        pl.semaphore_wait(barrier_sem, 1)

        @pl.when(dst_y == my_y)
        def _():
            copy = pltpu.make_async_copy(x_ref, out_ref, send_sem)
            copy.start()
            copy.wait()

        @pl.when(dst_y != my_y)
        def _():
            rdma = pltpu.make_async_remote_copy(
                src_ref=x_ref,
                dst_ref=out_ref,
                send_sem=send_sem,
                recv_sem=recv_sem,
                device_id=(my_x, dst_y, my_z),
                device_id_type=pl.DeviceIdType.MESH,
            )
            rdma.start()
            rdma.wait()

    return pl.pallas_call(
        body,
        out_shape=jax.ShapeDtypeStruct(x.shape, x.dtype),
        in_specs=[
            pl.BlockSpec(memory_space=pl.ANY),
            pl.BlockSpec(memory_space=pltpu.SMEM),
        ],
        out_specs=pl.BlockSpec(memory_space=pl.ANY),
        scratch_shapes=[
            pltpu.SemaphoreType.DMA,
            pltpu.SemaphoreType.DMA,
        ],
        compiler_params=pltpu.CompilerParams(collective_id=0),
    )(x, pi)
